# Initial kernel scaffold; baseline (speedup 1.0000x reference)
import jax
import jax.numpy as jnp
from jax.experimental import pallas as pl


def kernel(x, edge_index, edge_attr, batch, fp_x, cluster_x, params):
    raise NotImplementedError("write your pallas kernel here")


# speedup vs baseline: 1.2283x; 1.2283x over previous
import functools
import jax
import jax.numpy as jnp
from jax import lax
from jax.experimental import pallas as pl
from jax.experimental.pallas import tpu as pltpu

N_NODES = 50000
N_EDGES = 800000
NBLK = 2000
EBLK = 8000
H, C = 4, 32
HC = H * C

_SQRT2 = 1.4142135623730951


def _gelu(t):
    return 0.5 * t * (1.0 + lax.erf(t / _SQRT2))


def _proj_body(x_ref, wl_ref, bl_ref, wr_ref, br_ref, xl_ref, xr_ref):
    x = x_ref[...]
    xl_ref[...] = jnp.dot(x, wl_ref[...], preferred_element_type=jnp.float32) + bl_ref[...]
    xr_ref[...] = jnp.dot(x, wr_ref[...], preferred_element_type=jnp.float32) + br_ref[...]


def _edge_body(gl_ref, gr_ref, ea_ref, we_ref, a_ref, logit_ref, gmax_ref, easum_ref):
    i = pl.program_id(0)
    ep = jnp.dot(ea_ref[...], we_ref[...], preferred_element_type=jnp.float32)
    z = gl_ref[...] + gr_ref[...] + ep
    z = jnp.where(z >= 0.0, z, 0.2 * z)
    logits = jnp.dot(z, a_ref[...], preferred_element_type=jnp.float32)
    logit_ref[...] = logits
    bmax = jnp.max(logits, axis=0, keepdims=True)
    bsum = jnp.sum(ea_ref[...], axis=0, keepdims=True)

    @pl.when(i == 0)
    def _():
        gmax_ref[...] = jnp.full_like(gmax_ref, -jnp.inf)
        easum_ref[...] = jnp.zeros_like(easum_ref)

    gmax_ref[...] = jnp.maximum(gmax_ref[...], bmax)
    easum_ref[...] = easum_ref[...] + bsum


def _w_body(logit_ref, gmax_ref, w_ref):
    w_ref[...] = jnp.exp(logit_ref[...] - gmax_ref[...])


def _combine_body(nume_ref, dene_ref, xl_ref, xr_ref, easum_ref, we_ref,
                  a_ref, gmax_ref, r_ref, bias_ref,
                  pre_ref, bnsum_ref, bnsq_ref):
    i = pl.program_id(0)
    epm = jnp.dot(easum_ref[...] * (1.0 / N_EDGES), we_ref[...],
                  preferred_element_type=jnp.float32)
    z = xl_ref[...] + xr_ref[...] + epm
    z = jnp.where(z >= 0.0, z, 0.2 * z)
    logit_s = jnp.dot(z, a_ref[...], preferred_element_type=jnp.float32)
    w_s = jnp.exp(logit_s - gmax_ref[...])
    ws_wide = jnp.dot(w_s, r_ref[...], preferred_element_type=jnp.float32)
    den = dene_ref[...] + w_s
    den_wide = jnp.dot(den, r_ref[...], preferred_element_type=jnp.float32)
    pre = (nume_ref[...] + ws_wide * xl_ref[...]) / den_wide + bias_ref[...]
    pre_ref[...] = pre

    @pl.when(i == 0)
    def _():
        bnsum_ref[...] = jnp.zeros_like(bnsum_ref)
        bnsq_ref[...] = jnp.zeros_like(bnsq_ref)

    bnsum_ref[...] = bnsum_ref[...] + jnp.sum(pre, axis=0, keepdims=True)
    bnsq_ref[...] = bnsq_ref[...] + jnp.sum(pre * pre, axis=0, keepdims=True)


def _pool_body(pre_ref, bnsum_ref, bnsq_ref, batch_ref, g_ref, b_ref,
               gsum_ref, cnt_ref):
    i = pl.program_id(0)
    mu = bnsum_ref[...] * (1.0 / N_NODES)
    var = bnsq_ref[...] * (1.0 / N_NODES) - mu * mu
    h4 = (pre_ref[...] - mu) * lax.rsqrt(var + 1e-5) * g_ref[...] + b_ref[...]
    h4 = _gelu(h4)
    bblk = batch_ref[0]
    gids = lax.broadcasted_iota(jnp.int32, (256, 1), 0)
    onehot_t = (gids == bblk).astype(jnp.float32)

    @pl.when(i == 0)
    def _():
        gsum_ref[...] = jnp.zeros_like(gsum_ref)
        cnt_ref[...] = jnp.zeros_like(cnt_ref)

    gsum_ref[...] = gsum_ref[...] + jnp.dot(onehot_t, h4,
                                            preferred_element_type=jnp.float32)
    cnt_ref[...] = cnt_ref[...] + jnp.sum(onehot_t, axis=1, keepdims=True)


def _bn_cols(t, g, b):
    mu = jnp.mean(t, axis=0, keepdims=True)
    var = jnp.mean(t * t, axis=0, keepdims=True) - mu * mu
    return (t - mu) * lax.rsqrt(var + 1e-5) * g + b


def _head_body(gsum_ref, cnt_ref, fpx_ref, fpw_ref, fpb_ref, bnfpg_ref,
               bnfpb_ref, clus_ref, wg_ref, wf_ref, wc_ref, fc1b_ref,
               bnfg_ref, bnfb_ref, fc2w_ref, fc2b_ref, out_ref):
    cnt = jnp.maximum(cnt_ref[...][:, 0:1], 1.0)
    gat_emb = gsum_ref[...] / cnt
    fp1 = jnp.dot(fpx_ref[...], fpw_ref[...], preferred_element_type=jnp.float32) + fpb_ref[...]
    fp_emb = _gelu(_bn_cols(fp1, bnfpg_ref[...], bnfpb_ref[...]))
    h = (jnp.dot(gat_emb, wg_ref[...], preferred_element_type=jnp.float32)
         + jnp.dot(fp_emb, wf_ref[...], preferred_element_type=jnp.float32)
         + jnp.dot(clus_ref[...], wc_ref[...], preferred_element_type=jnp.float32)
         + fc1b_ref[...])
    h = _gelu(_bn_cols(h, bnfg_ref[...], bnfb_ref[...]))
    out_ref[...] = jnp.dot(h, fc2w_ref[...], preferred_element_type=jnp.float32) + fc2b_ref[...]


def kernel(x, edge_index, edge_attr, batch, fp_x, cluster_x, params):
    p = params
    f32 = jnp.float32
    src, dst = edge_index[0], edge_index[1]

    att = p['g4_att']
    eye_h = jnp.eye(H, dtype=f32)
    a_mat = (att[:, None, :] * eye_h[:, :, None]).reshape(H, HC).T
    r_mat = jnp.repeat(eye_h, C, axis=1)

    xl, xr = pl.pallas_call(
        _proj_body,
        grid=(N_NODES // NBLK,),
        in_specs=[
            pl.BlockSpec((NBLK, 54), lambda i: (i, 0)),
            pl.BlockSpec((54, HC), lambda i: (0, 0)),
            pl.BlockSpec((1, HC), lambda i: (0, 0)),
            pl.BlockSpec((54, HC), lambda i: (0, 0)),
            pl.BlockSpec((1, HC), lambda i: (0, 0)),
        ],
        out_specs=[
            pl.BlockSpec((NBLK, HC), lambda i: (i, 0)),
            pl.BlockSpec((NBLK, HC), lambda i: (i, 0)),
        ],
        out_shape=[
            jax.ShapeDtypeStruct((N_NODES, HC), f32),
            jax.ShapeDtypeStruct((N_NODES, HC), f32),
        ],
    )(x, p['g4_Wl'], p['g4_bl'][None, :], p['g4_Wr'], p['g4_br'][None, :])

    gl = jnp.take(xl, src, axis=0)
    gr = jnp.take(xr, dst, axis=0)

    logits, gmax, easum = pl.pallas_call(
        _edge_body,
        grid=(N_EDGES // EBLK,),
        in_specs=[
            pl.BlockSpec((EBLK, HC), lambda i: (i, 0)),
            pl.BlockSpec((EBLK, HC), lambda i: (i, 0)),
            pl.BlockSpec((EBLK, 12), lambda i: (i, 0)),
            pl.BlockSpec((12, HC), lambda i: (0, 0)),
            pl.BlockSpec((HC, H), lambda i: (0, 0)),
        ],
        out_specs=[
            pl.BlockSpec((EBLK, H), lambda i: (i, 0)),
            pl.BlockSpec((1, H), lambda i: (0, 0)),
            pl.BlockSpec((1, 12), lambda i: (0, 0)),
        ],
        out_shape=[
            jax.ShapeDtypeStruct((N_EDGES, H), f32),
            jax.ShapeDtypeStruct((1, H), f32),
            jax.ShapeDtypeStruct((1, 12), f32),
        ],
    )(gl, gr, edge_attr, p['g4_We'], a_mat)

    w = pl.pallas_call(
        _w_body,
        grid=(N_EDGES // EBLK,),
        in_specs=[
            pl.BlockSpec((EBLK, H), lambda i: (i, 0)),
            pl.BlockSpec((1, H), lambda i: (0, 0)),
        ],
        out_specs=pl.BlockSpec((EBLK, H), lambda i: (i, 0)),
        out_shape=jax.ShapeDtypeStruct((N_EDGES, H), f32),
    )(logits, gmax)

    num_e = jax.ops.segment_sum(w[:, :, None] * gl.reshape(N_EDGES, H, C),
                                dst, num_segments=N_NODES).reshape(N_NODES, HC)
    den_e = jax.ops.segment_sum(w, dst, num_segments=N_NODES)

    pre4, bnsum, bnsq = pl.pallas_call(
        _combine_body,
        grid=(N_NODES // NBLK,),
        in_specs=[
            pl.BlockSpec((NBLK, HC), lambda i: (i, 0)),
            pl.BlockSpec((NBLK, H), lambda i: (i, 0)),
            pl.BlockSpec((NBLK, HC), lambda i: (i, 0)),
            pl.BlockSpec((NBLK, HC), lambda i: (i, 0)),
            pl.BlockSpec((1, 12), lambda i: (0, 0)),
            pl.BlockSpec((12, HC), lambda i: (0, 0)),
            pl.BlockSpec((HC, H), lambda i: (0, 0)),
            pl.BlockSpec((1, H), lambda i: (0, 0)),
            pl.BlockSpec((H, HC), lambda i: (0, 0)),
            pl.BlockSpec((1, HC), lambda i: (0, 0)),
        ],
        out_specs=[
            pl.BlockSpec((NBLK, HC), lambda i: (i, 0)),
            pl.BlockSpec((1, HC), lambda i: (0, 0)),
            pl.BlockSpec((1, HC), lambda i: (0, 0)),
        ],
        out_shape=[
            jax.ShapeDtypeStruct((N_NODES, HC), f32),
            jax.ShapeDtypeStruct((1, HC), f32),
            jax.ShapeDtypeStruct((1, HC), f32),
        ],
    )(num_e, den_e, xl, xr, easum, p['g4_We'], a_mat, gmax, r_mat,
      p['g4_bias'][None, :])

    batch3 = batch.reshape(N_NODES // NBLK, 1, NBLK)
    gsum, cnt = pl.pallas_call(
        _pool_body,
        grid=(N_NODES // NBLK,),
        in_specs=[
            pl.BlockSpec((NBLK, HC), lambda i: (i, 0)),
            pl.BlockSpec((1, HC), lambda i: (0, 0)),
            pl.BlockSpec((1, HC), lambda i: (0, 0)),
            pl.BlockSpec((1, 1, NBLK), lambda i: (i, 0, 0)),
            pl.BlockSpec((1, HC), lambda i: (0, 0)),
            pl.BlockSpec((1, HC), lambda i: (0, 0)),
        ],
        out_specs=[
            pl.BlockSpec((256, HC), lambda i: (0, 0)),
            pl.BlockSpec((256, 1), lambda i: (0, 0)),
        ],
        out_shape=[
            jax.ShapeDtypeStruct((256, HC), f32),
            jax.ShapeDtypeStruct((256, 1), f32),
        ],
    )(pre4, bnsum, bnsq, batch3, p['bn4_g'][None, :], p['bn4_b'][None, :])

    out = pl.pallas_call(
        _head_body,
        in_specs=[pl.BlockSpec(s, lambda: (0, 0)) for s in [
            (256, HC), (256, 1), (256, 3387), (3387, 64), (1, 64), (1, 64),
            (1, 64), (256, 10), (HC, 32), (64, 32), (10, 32), (1, 32),
            (1, 32), (1, 32), (32, 1), (1, 1),
        ]],
        out_specs=pl.BlockSpec((256, 1), lambda: (0, 0)),
        out_shape=jax.ShapeDtypeStruct((256, 1), f32),
    )(gsum, cnt, fp_x, p['fp_W'], p['fp_b'][None, :], p['bnfp_g'][None, :],
      p['bnfp_b'][None, :], cluster_x, p['fc1_W'][:HC], p['fc1_W'][HC:HC + 64],
      p['fc1_W'][HC + 64:], p['fc1_b'][None, :], p['bnf_g'][None, :],
      p['bnf_b'][None, :], p['fc2_W'], p['fc2_b'][None, :])

    return out
